# baseline (speedup 1.0000x reference)
import jax
import jax.numpy as jnp
from jax import lax
from jax.experimental import pallas as pl
from jax.experimental.pallas import tpu as pltpu
from jax.experimental.pallas import tpu_sc as plsc

_INFO = plsc.get_sparse_core_info()
_NUM_WORKERS = _INFO.num_cores * _INFO.num_subcores


def _copy_body(rows_per_worker, w_hbm, out_hbm):
    wid = lax.axis_index("s") * _INFO.num_cores + lax.axis_index("c")
    base = wid * rows_per_worker
    pltpu.sync_copy(
        w_hbm.at[pl.ds(base, rows_per_worker)],
        out_hbm.at[pl.ds(base, rows_per_worker)],
    )


def kernel(x, emb_weight):
    seq = x.shape[1]
    dim = emb_weight.shape[1]
    rows_per_worker = seq // _NUM_WORKERS
    mesh = plsc.VectorSubcoreMesh(core_axis_name="c", subcore_axis_name="s")
    out = pl.kernel(
        lambda w, o: _copy_body(rows_per_worker, w, o),
        out_type=jax.ShapeDtypeStruct((seq, dim), emb_weight.dtype),
        mesh=mesh,
    )(emb_weight)
    return out[None]


# speedup vs baseline: 24.1947x; 24.1947x over previous
import jax
import jax.numpy as jnp
from jax import lax
from jax.experimental import pallas as pl
from jax.experimental.pallas import tpu as pltpu
from jax.experimental.pallas import tpu_sc as plsc

_INFO = plsc.get_sparse_core_info()
_NUM_WORKERS = _INFO.num_cores * _INFO.num_subcores

_NBUF = 4
_CHUNK_ROWS = 16


def _copy_body(rows_per_worker, dim, w_hbm, out_hbm, buf, sems_in, sems_out):
    wid = lax.axis_index("s") * _INFO.num_cores + lax.axis_index("c")
    base = wid * rows_per_worker
    nch = rows_per_worker // _CHUNK_ROWS

    def in_copy(c, b):
        return pltpu.make_async_copy(
            w_hbm.at[pl.ds(base + c * _CHUNK_ROWS, _CHUNK_ROWS)],
            buf.at[b],
            sems_in.at[b],
        )

    def out_copy(c, b):
        return pltpu.make_async_copy(
            buf.at[b],
            out_hbm.at[pl.ds(base + c * _CHUNK_ROWS, _CHUNK_ROWS)],
            sems_out.at[b],
        )

    for c in range(min(_NBUF, nch)):
        in_copy(c, c).start()
    for c in range(nch):
        b = c % _NBUF
        in_copy(c, b).wait()
        out_copy(c, b).start()
        nxt = c + _NBUF
        if nxt < nch:
            out_copy(c, b).wait()
            in_copy(nxt, b).start()
        else:
            out_copy(c, b).wait()


def kernel(x, emb_weight):
    seq = x.shape[1]
    dim = emb_weight.shape[1]
    rows_per_worker = seq // _NUM_WORKERS
    mesh = plsc.VectorSubcoreMesh(core_axis_name="c", subcore_axis_name="s")
    out = pl.kernel(
        lambda w, o, buf, si, so: _copy_body(
            rows_per_worker, dim, w, o, buf, si, so
        ),
        out_type=jax.ShapeDtypeStruct((seq, dim), emb_weight.dtype),
        mesh=mesh,
        scratch_types=[
            pltpu.VMEM((_NBUF, _CHUNK_ROWS, dim), jnp.float32),
            pltpu.SemaphoreType.DMA((_NBUF,)),
            pltpu.SemaphoreType.DMA((_NBUF,)),
        ],
    )(emb_weight)
    return out[None]
